# Initial kernel scaffold; baseline (speedup 1.0000x reference)
import jax
import jax.numpy as jnp
from jax.experimental import pallas as pl


def kernel(x, embed, edge_index, noise, tmp, W1, b1, W2, b2, W_gnn):
    raise NotImplementedError("write your pallas kernel here")


# speedup vs baseline: 1.8699x; 1.8699x over previous
import functools

import jax
import jax.numpy as jnp
from jax import lax
from jax.experimental import pallas as pl
from jax.experimental.pallas import tpu as pltpu
from jax.experimental.pallas import tpu_sc as plsc

NC = 2
NS = 16
LW = 16

_SC_PARAMS = pltpu.CompilerParams(use_tc_tiling_on_sc=False)


def _prep_body(embed, x, w1a, w1b, b1, wg, nz, rowb, colb,
               u, v, xw, nl, key, rkey, *, n_nodes):
    u[...] = jnp.dot(embed[...], w1a[...],
                     preferred_element_type=jnp.float32) + b1[...]
    v[...] = jnp.dot(embed[...], w1b[...], preferred_element_type=jnp.float32)
    xw[...] = jnp.dot(x[...], wg[...], preferred_element_type=jnp.float32)
    z = jnp.clip(nz[...], 1e-6, 1.0 - 1e-6)
    nl[...] = jnp.log(z) - jnp.log(1.0 - z)
    key[...] = rowb[...] * n_nodes + colb[...]
    rkey[...] = colb[...] * n_nodes + rowb[...]


def _gate_body(b2s, tmps, g1, g2, nlc, w2r, o):
    h = jnp.maximum(g1[...] + g2[...], 0.0)
    la = jnp.sum(h * w2r[...], axis=1, keepdims=True)
    o[...] = jax.nn.sigmoid((nlc[...] + la + b2s[0, 0]) / tmps[0, 0])


def _scale_body(vals, xwc, z):
    z[...] = vals[...] * xwc[...]


def _softmax_body(outp, res):
    o = outp[0] + outp[1]
    m = jnp.max(o, axis=0, keepdims=True)
    e = jnp.exp(o - m)
    res[...] = e / jnp.sum(e, axis=0, keepdims=True)


def _sc_gather_body(u_h, v_h, xw_h, row_h, col_h, g1_h, g2_h, xwc_h,
                    idx_v, buf_v, buf16_v, sem, *, epw, ch):
    wid = lax.axis_index("s") * NC + lax.axis_index("c")
    base = wid * epw

    @pl.loop(0, epw, step=ch)
    def _(off):
        b = base + off
        pltpu.sync_copy(row_h.at[pl.ds(b, ch)], idx_v)
        pltpu.async_copy(u_h.at[idx_v], buf_v, sem).wait()
        pltpu.sync_copy(buf_v, g1_h.at[pl.ds(b, ch)])
        pltpu.sync_copy(col_h.at[pl.ds(b, ch)], idx_v)
        pltpu.async_copy(v_h.at[idx_v], buf_v, sem).wait()
        pltpu.sync_copy(buf_v, g2_h.at[pl.ds(b, ch)])
        pltpu.async_copy(xw_h.at[idx_v], buf16_v, sem).wait()
        pltpu.sync_copy(buf16_v, xwc_h.at[pl.ds(b, ch)])


def _sc_ids_body(key_h, t_h, kv, dv, sem, *, epw):
    wid = lax.axis_index("s") * NC + lax.axis_index("c")
    base = wid * epw
    pltpu.sync_copy(key_h.at[pl.ds(base, epw)], kv)

    @pl.loop(0, epw, step=LW)
    def _(i):
        dv[pl.ds(i, LW)] = lax.iota(jnp.int32, LW) + (base + i)

    pltpu.sync_copy(dv, t_h.at[kv])


def _sc_vals_body(t_h, key_h, rkey_h, gate_h, vals_h,
                  kv, rv, cb, wb, g2b, kgb, gv, f1, f2, vv, s_sh, sem,
                  *, epw, n_edges):
    core = lax.axis_index("c")
    sid = lax.axis_index("s")

    @pl.when(core == 0)
    def _():
        base = sid * epw
        pltpu.sync_copy(key_h.at[pl.ds(base, epw)], kv)
        pltpu.sync_copy(rkey_h.at[pl.ds(base, epw)], rv)
        pltpu.sync_copy(gate_h.at[pl.ds(base, epw)], gv)

        pltpu.sync_copy(t_h.at[kv], wb)
        pltpu.sync_copy(t_h.at[rv], g2b)

        @pl.loop(0, epw, step=LW)
        def _(i):
            s = pl.ds(i, LW)
            cb[s] = jnp.minimum(jnp.maximum(g2b[s], 0), n_edges - 1)

        pltpu.sync_copy(key_h.at[cb], kgb)

        @pl.loop(0, epw, step=LW)
        def _(i):
            vv[pl.ds(i, LW)] = jnp.zeros((LW,), jnp.float32)

        pltpu.sync_copy(vv, s_sh.at[pl.ds(base, epw)])
        plsc.subcore_barrier()

        pltpu.sync_copy(gv, s_sh.at[wb], add=True)
        plsc.subcore_barrier()

        pltpu.sync_copy(s_sh.at[wb], f1)
        pltpu.sync_copy(s_sh.at[cb], f2)

        @pl.loop(0, epw, step=LW)
        def _(i):
            s = pl.ds(i, LW)
            valid = (g2b[s] == cb[s]) & (kgb[s] == rv[s])
            mcr = jnp.where(valid, f2[s], 0.0)
            val = 0.5 * (f1[s] + mcr)
            vv[s] = jnp.where(kv[s] == rv[s], 0.0, val)

        pltpu.sync_copy(vv, vals_h.at[pl.ds(base, epw)])


def _sc_segsum_body(row_h, z_h, z0_h, outp_h,
                    row_v, zbuf_v, s2_sh, sem, *, epw, ch, rows_per_sub):
    core = lax.axis_index("c")
    sid = lax.axis_index("s")
    wid = sid * NC + core
    base = wid * epw
    r0 = sid * rows_per_sub

    pltpu.sync_copy(z0_h.at[pl.ds(r0, rows_per_sub)],
                    s2_sh.at[pl.ds(r0, rows_per_sub)])
    plsc.subcore_barrier()

    @pl.loop(0, epw, step=ch)
    def _(off):
        b = base + off
        pltpu.sync_copy(row_h.at[pl.ds(b, ch)], row_v)
        pltpu.sync_copy(z_h.at[pl.ds(b, ch)], zbuf_v)
        pltpu.sync_copy(zbuf_v, s2_sh.at[row_v], add=True)

    plsc.subcore_barrier()
    pltpu.sync_copy(s2_sh.at[pl.ds(r0, rows_per_sub)],
                    outp_h.at[core, pl.ds(r0, rows_per_sub)])


def kernel(x, embed, edge_index, noise, tmp, W1, b1, W2, b2, W_gnn):
    N, D = x.shape
    E = edge_index.shape[1]
    H = W1.shape[1]
    C = W_gnn.shape[1]
    f32, i32 = jnp.float32, jnp.int32

    row = edge_index[0].astype(i32)
    col = edge_index[1].astype(i32)
    noise2 = noise.reshape(E // 128, 128)
    row2 = row.reshape(E // 128, 128)
    col2 = col.reshape(E // 128, 128)
    mesh = plsc.VectorSubcoreMesh(core_axis_name="c", subcore_axis_name="s",
                                  num_cores=NC, num_subcores=NS)

    u, v, xw, nl2, key2, rkey2 = pl.pallas_call(
        functools.partial(_prep_body, n_nodes=N),
        out_shape=[
            jax.ShapeDtypeStruct((N, H), f32),
            jax.ShapeDtypeStruct((N, H), f32),
            jax.ShapeDtypeStruct((N, C), f32),
            jax.ShapeDtypeStruct((E // 128, 128), f32),
            jax.ShapeDtypeStruct((E // 128, 128), i32),
            jax.ShapeDtypeStruct((E // 128, 128), i32),
        ],
    )(embed, x, W1[:D], W1[D:], b1.reshape(1, H), W_gnn, noise2, row2, col2)
    nl = nl2.reshape(E, 1)
    key = key2.reshape(E)
    rkey = rkey2.reshape(E)

    epw = E // (NC * NS)
    ch = 1000
    g1, g2, xwc = pl.kernel(
        functools.partial(_sc_gather_body, epw=epw, ch=ch),
        out_type=[
            jax.ShapeDtypeStruct((E, H), f32),
            jax.ShapeDtypeStruct((E, H), f32),
            jax.ShapeDtypeStruct((E, C), f32),
        ],
        mesh=mesh,
        compiler_params=_SC_PARAMS,
        scratch_types=[
            pltpu.VMEM((ch,), i32),
            pltpu.VMEM((ch, H), f32),
            pltpu.VMEM((ch, C), f32),
            pltpu.SemaphoreType.DMA,
        ],
    )(u, v, xw, row, col)

    be = 2000
    gate = pl.pallas_call(
        _gate_body,
        grid=(E // be,),
        in_specs=[
            pl.BlockSpec(memory_space=pltpu.SMEM),
            pl.BlockSpec(memory_space=pltpu.SMEM),
            pl.BlockSpec((be, H), lambda i: (i, 0)),
            pl.BlockSpec((be, H), lambda i: (i, 0)),
            pl.BlockSpec((be, 1), lambda i: (i, 0)),
            pl.BlockSpec((1, H), lambda i: (0, 0)),
        ],
        out_specs=pl.BlockSpec((be, 1), lambda i: (i, 0)),
        out_shape=jax.ShapeDtypeStruct((E, 1), f32),
    )(b2.reshape(1, 1), tmp.reshape(1, 1).astype(f32), g1, g2, nl,
      W2.reshape(1, H))

    t_tab = pl.kernel(
        functools.partial(_sc_ids_body, epw=epw),
        out_type=jax.ShapeDtypeStruct((N * N,), i32),
        mesh=mesh,
        compiler_params=_SC_PARAMS,
        scratch_types=[
            pltpu.VMEM((epw,), i32),
            pltpu.VMEM((epw,), i32),
            pltpu.SemaphoreType.DMA,
        ],
    )(key)

    epw2 = E // NS
    vals = pl.kernel(
        functools.partial(_sc_vals_body, epw=epw2, n_edges=E),
        out_type=jax.ShapeDtypeStruct((E,), f32),
        mesh=mesh,
        compiler_params=_SC_PARAMS,
        scratch_types=[
            pltpu.VMEM((epw2,), i32),
            pltpu.VMEM((epw2,), i32),
            pltpu.VMEM((epw2,), i32),
            pltpu.VMEM((epw2,), i32),
            pltpu.VMEM((epw2,), i32),
            pltpu.VMEM((epw2,), i32),
            pltpu.VMEM((epw2,), f32),
            pltpu.VMEM((epw2,), f32),
            pltpu.VMEM((epw2,), f32),
            pltpu.VMEM((epw2,), f32),
            pltpu.VMEM_SHARED((E,), f32),
            pltpu.SemaphoreType.DMA,
        ],
    )(t_tab, key, rkey, gate.reshape(E))

    z = pl.pallas_call(
        _scale_body,
        grid=(E // be,),
        in_specs=[
            pl.BlockSpec((be, 1), lambda i: (i, 0)),
            pl.BlockSpec((be, C), lambda i: (i, 0)),
        ],
        out_specs=pl.BlockSpec((be, C), lambda i: (i, 0)),
        out_shape=jax.ShapeDtypeStruct((E, C), f32),
    )(vals.reshape(E, 1), xwc)

    rows_per_sub = N // NS
    outp = pl.kernel(
        functools.partial(_sc_segsum_body, epw=epw, ch=ch,
                          rows_per_sub=rows_per_sub),
        out_type=jax.ShapeDtypeStruct((NC, N, C), f32),
        mesh=mesh,
        compiler_params=_SC_PARAMS,
        scratch_types=[
            pltpu.VMEM((ch,), i32),
            pltpu.VMEM((ch, C), f32),
            pltpu.VMEM_SHARED((N, C), f32),
            pltpu.SemaphoreType.DMA,
        ],
    )(row, z, jnp.zeros((N, C), f32))

    res = pl.pallas_call(
        _softmax_body,
        out_shape=jax.ShapeDtypeStruct((N, C), f32),
    )(outp)
    return res
